# baseline (speedup 1.0000x reference)
import jax
import jax.numpy as jnp
import numpy as np
from jax.experimental import pallas as pl
from jax.experimental.pallas import tpu as pltpu

_T = 128
_D = 64
_TOPK = 8
_MAXR = 255.0

_BB = 64
_CH = 64


def _make_d8m():
    i = np.arange(_T)
    d = np.abs(i[:, None] - i[None, :]).astype(np.float32)
    decay = np.float32(1.0) - (np.float32(0.1) * d) / np.float32(128.0)
    tril = i[None, :] <= i[:, None]
    return np.where(tril, decay * np.float32(0.125),
                    np.float32(0.0)).astype(np.float32)


_D8M = _make_d8m()


def _head_body(x_ref, wq_ref, wk_ref, wv_ref, d8m_ref, g_ref, out_ref,
               s_scr, w_scr):
    x = x_ref[...].reshape(_BB * _T, _D)
    q = jnp.dot(x, wq_ref[...], preferred_element_type=jnp.float32)
    k = jnp.dot(x, wk_ref[...], preferred_element_type=jnp.float32)
    v = jnp.dot(x, wv_ref[...], preferred_element_type=jnp.float32)
    q = q.reshape(_BB, _T, _D)
    k = k.reshape(_BB, _T, _D)
    v = v.reshape(_BB, _T, _D) * (1.0 / g_ref[0, 0])

    s = jax.lax.dot_general(
        q, k, (((2,), (2,)), ((0,), (0,))),
        preferred_element_type=jnp.float32)
    s_scr[...] = s.reshape(_BB * _T, _T)

    def chunk(i, carry):
        base = i * _CH
        tile = jnp.maximum(s_scr[pl.ds(base, _CH), :], 0.0)
        tile = tile * d8m_ref[pl.ds((i % (_T // _CH)) * _CH, _CH), :]

        mean = jnp.mean(tile, axis=-1, keepdims=True)
        sumsq = jnp.sum(tile * tile, axis=-1, keepdims=True)
        var = jnp.maximum(sumsq - mean * mean * _T, 0.0) / (_T - 1)
        sigma = jnp.sqrt(var)

        thresh = jnp.max(tile, axis=-1, keepdims=True)
        m = thresh
        for _ in range(_TOPK - 1):
            thresh = jnp.max(tile * (tile < thresh), axis=-1, keepdims=True)

        r = _MAXR / (jnp.maximum(m, sigma) + 1e-6)
        norm = jnp.floor(tile * r)
        w_scr[pl.ds(base, _CH), :] = jnp.where(
            tile >= thresh, norm - jnp.where(norm > 127.5, 256.0, 0.0), 0.0)
        return carry

    jax.lax.fori_loop(0, (_BB * _T) // _CH, chunk, 0, unroll=False)

    w = w_scr[...].reshape(_BB, _T, _T)
    out_ref[...] = jax.lax.dot_general(
        w, v, (((2,), (1,)), ((0,), (0,))),
        preferred_element_type=jnp.float32)


def kernel(x, Wk, Wq, Wv, gamma):
    b, t, d = x.shape
    g = jnp.reshape(gamma, (1, 1)).astype(jnp.float32)
    return pl.pallas_call(
        _head_body,
        grid=(b // _BB,),
        in_specs=[
            pl.BlockSpec((_BB, t, d), lambda i: (i, 0, 0)),
            pl.BlockSpec((d, d), lambda i: (0, 0)),
            pl.BlockSpec((d, d), lambda i: (0, 0)),
            pl.BlockSpec((d, d), lambda i: (0, 0)),
            pl.BlockSpec((t, t), lambda i: (0, 0)),
            pl.BlockSpec((1, 1), lambda i: (0, 0)),
        ],
        out_specs=pl.BlockSpec((_BB, t, d), lambda i: (i, 0, 0)),
        out_shape=jax.ShapeDtypeStruct((b, t, d), jnp.float32),
        scratch_shapes=[
            pltpu.VMEM((_BB * _T, _T), jnp.float32),
            pltpu.VMEM((_BB * _T, _T), jnp.float32),
        ],
    )(x, Wq, Wk, Wv, jnp.asarray(_D8M), g)


# speedup vs baseline: 4.6316x; 4.6316x over previous
import jax
import jax.numpy as jnp
import numpy as np
from jax.experimental import pallas as pl

_T = 128
_D = 64
_TOPK = 8
_MAXR = 255.0

_BB = 64


def _make_d8m():
    i = np.arange(_T)
    d = np.abs(i[:, None] - i[None, :]).astype(np.float32)
    decay = np.float32(1.0) - (np.float32(0.1) * d) / np.float32(128.0)
    tril = i[None, :] <= i[:, None]
    return np.where(tril, decay * np.float32(0.125),
                    np.float32(0.0)).astype(np.float32)


_D8M = _make_d8m()


def _head_body(x_ref, wq_ref, wk_ref, wv_ref, d8m_ref, g_ref, out_ref):
    x = x_ref[...].reshape(_BB * _T, _D)
    q = jnp.dot(x, wq_ref[...], preferred_element_type=jnp.float32)
    k = jnp.dot(x, wk_ref[...], preferred_element_type=jnp.float32)
    v = jnp.dot(x, wv_ref[...], preferred_element_type=jnp.float32)
    q = q.reshape(_BB, _T, _D)
    k = k.reshape(_BB, _T, _D)
    v = v.reshape(_BB, _T, _D) * (1.0 / g_ref[0, 0])

    s = jax.lax.dot_general(
        q, k, (((2,), (2,)), ((0,), (0,))),
        preferred_element_type=jnp.float32)

    f = jnp.maximum(s, 0.0) * d8m_ref[...][None]

    mean = jnp.mean(f, axis=-1, keepdims=True)
    sumsq = jnp.sum(f * f, axis=-1, keepdims=True)
    var = jnp.maximum(sumsq - mean * mean * _T, 0.0) / (_T - 1)
    sigma = jnp.sqrt(var)

    thresh = jnp.max(f, axis=-1, keepdims=True)
    m = thresh
    for _ in range(_TOPK - 1):
        thresh = jnp.max(jnp.where(f < thresh, f, 0.0), axis=-1, keepdims=True)

    denom = jnp.maximum(m, sigma) + 1e-6
    r = _MAXR / denom
    norm = jnp.floor(f * r)
    w = jnp.where(f >= thresh, norm - jnp.where(norm > 127.5, 256.0, 0.0),
                  0.0)

    out_ref[...] = jax.lax.dot_general(
        w, v, (((2,), (1,)), ((0,), (0,))),
        preferred_element_type=jnp.float32)


def kernel(x, Wk, Wq, Wv, gamma):
    b, t, d = x.shape
    g = jnp.reshape(gamma, (1, 1)).astype(jnp.float32)
    return pl.pallas_call(
        _head_body,
        grid=(b // _BB,),
        in_specs=[
            pl.BlockSpec((_BB, t, d), lambda i: (i, 0, 0)),
            pl.BlockSpec((d, d), lambda i: (0, 0)),
            pl.BlockSpec((d, d), lambda i: (0, 0)),
            pl.BlockSpec((d, d), lambda i: (0, 0)),
            pl.BlockSpec((t, t), lambda i: (0, 0)),
            pl.BlockSpec((1, 1), lambda i: (0, 0)),
        ],
        out_specs=pl.BlockSpec((_BB, t, d), lambda i: (i, 0, 0)),
        out_shape=jax.ShapeDtypeStruct((b, t, d), jnp.float32),
    )(x, Wq, Wk, Wv, jnp.asarray(_D8M), g)


# speedup vs baseline: 5.2513x; 1.1338x over previous
import jax
import jax.numpy as jnp
import numpy as np
from jax.experimental import pallas as pl

_T = 128
_D = 64
_TOPK = 8
_MAXR = 255.0

_BB = 64


def _make_d8m():
    i = np.arange(_T)
    d = np.abs(i[:, None] - i[None, :]).astype(np.float32)
    decay = np.float32(1.0) - (np.float32(0.1) * d) / np.float32(128.0)
    tril = i[None, :] <= i[:, None]
    return np.where(tril, decay * np.float32(0.125),
                    np.float32(0.0)).astype(np.float32)


_D8M = _make_d8m()


def _head_body(x_ref, wq_ref, wk_ref, wv_ref, d8m_ref, g_ref, out_ref):
    x = x_ref[...].reshape(_BB * _T, _D)
    q = jnp.dot(x, wq_ref[...], preferred_element_type=jnp.float32)
    k = jnp.dot(x, wk_ref[...], preferred_element_type=jnp.float32)
    v = jnp.dot(x, wv_ref[...], preferred_element_type=jnp.float32)
    q = q.reshape(_BB, _T, _D)
    k = k.reshape(_BB, _T, _D)
    v = v.reshape(_BB, _T, _D) * (1.0 / g_ref[0, 0])

    s = jax.lax.dot_general(
        q, k, (((2,), (2,)), ((0,), (0,))),
        preferred_element_type=jnp.float32)

    f = jnp.maximum(s, 0.0) * d8m_ref[...][None]


    thresh = jnp.max(f, axis=-1, keepdims=True)
    m = thresh
    for _ in range(_TOPK - 1):
        thresh = jnp.max(jnp.where(f < thresh, f, 0.0), axis=-1, keepdims=True)

    denom = m + 1e-6
    r = _MAXR / denom
    norm = jnp.floor(f * r)
    w = jnp.where(f >= thresh, norm - jnp.where(norm > 127.5, 256.0, 0.0),
                  0.0)

    out_ref[...] = jax.lax.dot_general(
        w, v, (((2,), (1,)), ((0,), (0,))),
        preferred_element_type=jnp.float32)


def kernel(x, Wk, Wq, Wv, gamma):
    b, t, d = x.shape
    g = jnp.reshape(gamma, (1, 1)).astype(jnp.float32)
    return pl.pallas_call(
        _head_body,
        grid=(b // _BB,),
        in_specs=[
            pl.BlockSpec((_BB, t, d), lambda i: (i, 0, 0)),
            pl.BlockSpec((d, d), lambda i: (0, 0)),
            pl.BlockSpec((d, d), lambda i: (0, 0)),
            pl.BlockSpec((d, d), lambda i: (0, 0)),
            pl.BlockSpec((t, t), lambda i: (0, 0)),
            pl.BlockSpec((1, 1), lambda i: (0, 0)),
        ],
        out_specs=pl.BlockSpec((_BB, t, d), lambda i: (i, 0, 0)),
        out_shape=jax.ShapeDtypeStruct((b, t, d), jnp.float32),
    )(x, Wq, Wk, Wv, jnp.asarray(_D8M), g)
